# Initial kernel scaffold; baseline (speedup 1.0000x reference)
import jax
import jax.numpy as jnp
from jax.experimental import pallas as pl


def kernel(pcd_a, pcd_b, feat_a, feat_b, W_attn, key_W, key_b):
    raise NotImplementedError("write your pallas kernel here")


# speedup vs baseline: 7.1445x; 7.1445x over previous
import functools

import jax
import jax.numpy as jnp
import numpy as np
from jax import lax
from jax.experimental import pallas as pl
from jax.experimental.pallas import tpu as pltpu
from jax.experimental.pallas import tpu_sc as plsc

D = 256
H = 8
K = 16
DH = D // H
N = 4096
TEMPERATURE = 0.25


def _knn_body(q2_ref, b2_ref, mm_ref, idx_ref):
    q2 = q2_ref[...]
    b2 = b2_ref[...]
    mm = mm_ref[...]
    nb = b2.shape[1]
    neg = 2.0 * mm - (q2 + b2)
    iota = lax.broadcasted_iota(jnp.int32, neg.shape, 1)
    cols = []
    cur = neg
    for _ in range(K):
        m = jnp.max(cur, axis=1, keepdims=True)
        am = jnp.min(jnp.where(cur == m, iota, nb), axis=1, keepdims=True)
        cols.append(am)
        cur = jnp.where(iota == am, -jnp.inf, cur)
    idx_ref[...] = jnp.concatenate(cols, axis=1)


def _knn(query, base, *, interpret=False):
    bq = 256
    nq = query.shape[0]
    nb = base.shape[0]
    q2 = jnp.sum(query * query, axis=-1, keepdims=True)
    b2 = jnp.sum(base * base, axis=-1)[None, :]
    mm = query @ base.T
    return pl.pallas_call(
        _knn_body,
        grid=(nq // bq,),
        in_specs=[
            pl.BlockSpec((bq, 1), lambda i: (i, 0)),
            pl.BlockSpec((1, nb), lambda i: (0, 0)),
            pl.BlockSpec((bq, nb), lambda i: (i, 0)),
        ],
        out_specs=pl.BlockSpec((bq, K), lambda i: (i, 0)),
        out_shape=jax.ShapeDtypeStruct((nq, K), jnp.int32),
        interpret=interpret,
    )(q2, b2, mm)


def _proj_body(f_ref, wk_ref, wv_ref, kp_ref, vp_ref):
    f = f_ref[...]
    kp_ref[...] = jnp.dot(f, wk_ref[...], preferred_element_type=jnp.float32)
    vp_ref[...] = jnp.dot(f, wv_ref[...], preferred_element_type=jnp.float32)


def _proj(f, wk, wv, *, interpret=False):
    bn = 512
    n = f.shape[0]
    return pl.pallas_call(
        _proj_body,
        grid=(n // bn,),
        in_specs=[
            pl.BlockSpec((bn, D), lambda i: (i, 0)),
            pl.BlockSpec((D, D), lambda i: (0, 0)),
            pl.BlockSpec((D, D), lambda i: (0, 0)),
        ],
        out_specs=[
            pl.BlockSpec((bn, D), lambda i: (i, 0)),
            pl.BlockSpec((bn, D), lambda i: (i, 0)),
        ],
        out_shape=[
            jax.ShapeDtypeStruct((n, D), jnp.float32),
            jax.ShapeDtypeStruct((n, D), jnp.float32),
        ],
        interpret=interpret,
    )(f, wk, wv)


@functools.cache
def _make_gather(b_total):
    info = plsc.get_sparse_core_info()
    nw = info.num_cores * info.num_subcores
    b_per_w = b_total // nw
    ch = 128
    mesh = plsc.VectorSubcoreMesh(core_axis_name="c", subcore_axis_name="s")

    @functools.partial(
        pl.kernel,
        mesh=mesh,
        out_type=[
            jax.ShapeDtypeStruct((b_total, D), jnp.float32),
            jax.ShapeDtypeStruct((b_total, D), jnp.float32),
        ],
        scratch_types=[
            pltpu.VMEM((ch,), jnp.int32),
            pltpu.VMEM((ch, D), jnp.float32),
            pltpu.VMEM((ch, D), jnp.float32),
            pltpu.SemaphoreType.DMA,
            pltpu.SemaphoreType.DMA,
        ],
    )
    def g(kp_hbm, vp_hbm, idx_hbm, kn_hbm, vn_hbm, idx_v, krows_v, vrows_v,
          ksem, vsem):
        wid = lax.axis_index("s") * info.num_cores + lax.axis_index("c")
        base = wid * b_per_w
        for c in range(b_per_w // ch):
            off = base + c * ch
            pltpu.sync_copy(idx_hbm.at[pl.ds(off, ch)], idx_v)
            kcp = pltpu.async_copy(kp_hbm.at[idx_v], krows_v, ksem)
            vcp = pltpu.async_copy(vp_hbm.at[idx_v], vrows_v, vsem)
            kcp.wait()
            pltpu.sync_copy(krows_v, kn_hbm.at[pl.ds(off, ch)])
            vcp.wait()
            pltpu.sync_copy(vrows_v, vn_hbm.at[pl.ds(off, ch)])

    return g


def _gather(kp, vp, idx_flat):
    return _make_gather(idx_flat.shape[0])(kp, vp, idx_flat)


def _rolll(x, s):
    return pltpu.roll(x, D - s, 1)


def _head_sums(p):
    t = p + _rolll(p, 8)
    t = t + _rolll(p, 16)
    t = t + _rolll(p, 24)
    t = t + _rolll(t, 4)
    t = t + _rolll(t, 2)
    t = t + _rolll(t, 1)
    return t


def _halving(vs):
    vs = list(vs)
    while len(vs) > 1:
        h = len(vs) // 2
        vs = [vs[i] + vs[i + h] for i in range(h)]
    return vs[0]


def _attn_body(qf_ref, kn_ref, vn_ref, wq_ref, wo_ref, sel_ref, out_ref):
    qf = qf_ref[...]
    q = jnp.dot(qf, wq_ref[...], preferred_element_type=jnp.float32)
    sel = sel_ref[...]
    ls = []
    for k in range(K):
        t = _head_sums(q * kn_ref[k])
        l = jnp.dot(t, sel, precision=lax.Precision.HIGHEST,
                    preferred_element_type=jnp.float32) / np.sqrt(DH)
        ls.append(l)
    m = ls[0]
    for k in range(1, K):
        m = jnp.maximum(m, ls[k])
    es = [jnp.exp(l - m) for l in ls]
    s = _halving(es)
    o = (es[0] / s) * vn_ref[0]
    for k in range(1, K):
        o = o + (es[k] / s) * vn_ref[k]
    out_ref[...] = qf + jnp.dot(o, wo_ref[...], preferred_element_type=jnp.float32)


def _attn(qf, kn, vn, wq, wo, sel, *, interpret=False):
    bn = 256
    n = qf.shape[0]
    return pl.pallas_call(
        _attn_body,
        grid=(n // bn,),
        in_specs=[
            pl.BlockSpec((bn, D), lambda i: (i, 0)),
            pl.BlockSpec((K, bn, D), lambda i: (0, i, 0)),
            pl.BlockSpec((K, bn, D), lambda i: (0, i, 0)),
            pl.BlockSpec((D, D), lambda i: (0, 0)),
            pl.BlockSpec((D, D), lambda i: (0, 0)),
            pl.BlockSpec((D, D), lambda i: (0, 0)),
        ],
        out_specs=pl.BlockSpec((bn, D), lambda i: (i, 0)),
        out_shape=jax.ShapeDtypeStruct((n, D), jnp.float32),
        interpret=interpret,
    )(qf, kn, vn, wq, wo, sel)


def _head_body(f_ref, kw_ref, kb_ref, out_ref):
    logits = (jnp.dot(f_ref[...], kw_ref[...], preferred_element_type=jnp.float32)
              + kb_ref[...]) / TEMPERATURE
    out_ref[...] = jax.nn.sigmoid(logits)


def _head(f, kw, kb, *, interpret=False):
    n = f.shape[0]
    return pl.pallas_call(
        _head_body,
        grid=(1,),
        in_specs=[
            pl.BlockSpec((n, D), lambda i: (0, 0)),
            pl.BlockSpec((D, 1), lambda i: (0, 0)),
            pl.BlockSpec((1, 1), lambda i: (0, 0)),
        ],
        out_specs=pl.BlockSpec((n, 1), lambda i: (0, 0)),
        out_shape=jax.ShapeDtypeStruct((n, 1), jnp.float32),
        interpret=interpret,
    )(f, kw, kb.reshape(1, 1))


def _sel_mask():
    sel = np.zeros((D, D), np.float32)
    for j in range(D):
        sel[(j // DH) * DH, j] = 1.0
    return jnp.asarray(sel)


def _attn_layer(qf, kvf, idx_flat, wq, wk, wv, wo, sel):
    kp, vp = _proj(kvf, wk, wv)
    kn, vn = _gather(kp, vp, idx_flat)
    kn = kn.reshape(K, N, D)
    vn = vn.reshape(K, N, D)
    return _attn(qf, kn, vn, wq, wo, sel)


def kernel(pcd_a, pcd_b, feat_a, feat_b, W_attn, key_W, key_b):
    depth = W_attn.shape[0]
    idx_a2a = _knn(pcd_a, pcd_a).T.reshape(-1)
    idx_b2b = _knn(pcd_b, pcd_b).T.reshape(-1)
    idx_a2b = _knn(pcd_a, pcd_b).T.reshape(-1)
    idx_b2a = _knn(pcd_b, pcd_a).T.reshape(-1)
    sel = _sel_mask()
    fa, fb = feat_a, feat_b
    probs = []
    for i in range(depth):
        wq, wk, wv, wo = (W_attn[i, 0, j] for j in range(4))
        fa_new = _attn_layer(fa, fb, idx_a2b, wq, wk, wv, wo, sel)
        fb_new = _attn_layer(fb, fa, idx_b2a, wq, wk, wv, wo, sel)
        fa, fb = fa_new, fb_new
        wq, wk, wv, wo = (W_attn[i, 1, j] for j in range(4))
        fa = _attn_layer(fa, fa, idx_a2a, wq, wk, wv, wo, sel)
        fb = _attn_layer(fb, fb, idx_b2b, wq, wk, wv, wo, sel)
        wq, wk, wv, wo = (W_attn[i, 2, j] for j in range(4))
        if i == depth - 1:
            fa = _attn_layer(fa, fb, idx_a2b, wq, wk, wv, wo, sel)
        else:
            fa_new = _attn_layer(fa, fb, idx_a2b, wq, wk, wv, wo, sel)
            fb_new = _attn_layer(fb, fa, idx_b2a, wq, wk, wv, wo, sel)
            fa, fb = fa_new, fb_new
        probs.append(_head(fa, key_W[i], key_b[i]))
    return jnp.stack(probs, axis=0)
